# baseline (speedup 1.0000x reference)
import functools

import jax
import jax.numpy as jnp
from jax import lax
from jax.experimental import pallas as pl
from jax.experimental.pallas import tpu as pltpu
from jax.experimental.pallas import tpu_sc as plsc

N = 10000
D = 128
E = 320000
K = 2

NC = 2
NS = 16
NW = NC * NS
CH = 125
EW = E // NW
CPW = EW // CH
NP = 10240
RPT = NP // NS

RB = 1000
GRID = N // RB

_MESH = plsc.VectorSubcoreMesh(core_axis_name="c", subcore_axis_name="s")


@functools.partial(
    pl.kernel,
    out_type=jax.ShapeDtypeStruct((NC, NP, D), jnp.float32),
    mesh=_MESH,
    scratch_types=[
        pltpu.VMEM((CPW, CH), jnp.int32),
        pltpu.VMEM((CPW, CH), jnp.int32),
        pltpu.VMEM((CH, D), jnp.float32),
        pltpu.VMEM_SHARED((NP, D), jnp.float32),
        pltpu.SemaphoreType.DMA,
    ],
)
def _prop_kernel(u_hbm, src_hbm, dst_hbm, zeros_hbm, out_hbm,
                 srcv, dstv, buf, acc, gsem):
    c = lax.axis_index("c")
    s = lax.axis_index("s")
    w = s * NC + c
    pltpu.sync_copy(src_hbm.at[pl.ds(w * CPW, CPW)], srcv)
    pltpu.sync_copy(dst_hbm.at[pl.ds(w * CPW, CPW)], dstv)
    pltpu.sync_copy(zeros_hbm.at[pl.ds(s * RPT, RPT)],
                    acc.at[pl.ds(s * RPT, RPT)])
    plsc.subcore_barrier()

    def body(i, carry):
        pltpu.async_copy(u_hbm.at[srcv.at[i]], buf, gsem).wait()
        pltpu.sync_copy(buf, acc.at[dstv.at[i]], add=True)
        return carry

    lax.fori_loop(0, CPW, body, 0)
    plsc.subcore_barrier()
    pltpu.sync_copy(acc.at[pl.ds(s * RPT, RPT)],
                    out_hbm.at[c].at[pl.ds(s * RPT, RPT)])


def _norm_u0_body(dega_ref, degb_ref, feat_ref, norm_ref, u0_ref):
    deg = dega_ref[..., 0:1] + degb_ref[..., 0:1]
    nrm = lax.rsqrt(jnp.maximum(deg, 1.0))
    norm_ref[...] = nrm
    u0_ref[...] = feat_ref[...] * nrm


_norm_u0 = pl.pallas_call(
    _norm_u0_body,
    grid=(GRID,),
    in_specs=[
        pl.BlockSpec((RB, D), lambda i: (i, 0)),
        pl.BlockSpec((RB, D), lambda i: (i, 0)),
        pl.BlockSpec((RB, D), lambda i: (i, 0)),
    ],
    out_specs=[
        pl.BlockSpec((RB, 1), lambda i: (i, 0)),
        pl.BlockSpec((RB, D), lambda i: (i, 0)),
    ],
    out_shape=[
        jax.ShapeDtypeStruct((N, 1), jnp.float32),
        jax.ShapeDtypeStruct((N, D), jnp.float32),
    ],
)


def _combine_body(pa_ref, pb_ref, norm_ref, f_ref, u_ref):
    i = pl.program_id(0)
    nb = norm_ref[pl.ds(i * RB, RB), :]
    f = (pa_ref[...] + pb_ref[...]) * nb
    f_ref[...] = f
    u_ref[...] = f * nb


_combine = pl.pallas_call(
    _combine_body,
    grid=(GRID,),
    in_specs=[
        pl.BlockSpec((RB, D), lambda i: (i, 0)),
        pl.BlockSpec((RB, D), lambda i: (i, 0)),
        pl.BlockSpec((N, 1), lambda i: (0, 0)),
    ],
    out_specs=[
        pl.BlockSpec((RB, D), lambda i: (i, 0)),
        pl.BlockSpec((RB, D), lambda i: (i, 0)),
    ],
    out_shape=[
        jax.ShapeDtypeStruct((N, D), jnp.float32),
        jax.ShapeDtypeStruct((N, D), jnp.float32),
    ],
)


def _mm1_body(f0_ref, f1_ref, pa_ref, pb_ref, norm_ref, w_ref, b_ref,
              h_ref, u_ref):
    i = pl.program_id(0)
    nb = norm_ref[pl.ds(i * RB, RB), :]
    f2 = (pa_ref[...] + pb_ref[...]) * nb
    w = w_ref[...]
    h = jnp.dot(f0_ref[...], w[0:D], preferred_element_type=jnp.float32)
    h = h + jnp.dot(f1_ref[...], w[D:2 * D], preferred_element_type=jnp.float32)
    h = h + jnp.dot(f2, w[2 * D:3 * D], preferred_element_type=jnp.float32)
    h = h + b_ref[...]
    h_ref[...] = h
    u_ref[...] = h * nb


_mm1 = pl.pallas_call(
    _mm1_body,
    grid=(GRID,),
    in_specs=[
        pl.BlockSpec((RB, D), lambda i: (i, 0)),
        pl.BlockSpec((RB, D), lambda i: (i, 0)),
        pl.BlockSpec((RB, D), lambda i: (i, 0)),
        pl.BlockSpec((RB, D), lambda i: (i, 0)),
        pl.BlockSpec((N, 1), lambda i: (0, 0)),
        pl.BlockSpec(((K + 1) * D, D), lambda i: (0, 0)),
        pl.BlockSpec((1, D), lambda i: (0, 0)),
    ],
    out_specs=[
        pl.BlockSpec((RB, D), lambda i: (i, 0)),
        pl.BlockSpec((RB, D), lambda i: (i, 0)),
    ],
    out_shape=[
        jax.ShapeDtypeStruct((N, D), jnp.float32),
        jax.ShapeDtypeStruct((N, D), jnp.float32),
    ],
)


def _mm2_body(h1_ref, f1_ref, pa_ref, pb_ref, norm_ref, w_ref, b_ref,
              wfc_ref, bfc_ref, out_ref):
    i = pl.program_id(0)
    nb = norm_ref[pl.ds(i * RB, RB), :]
    f2 = (pa_ref[...] + pb_ref[...]) * nb
    w = w_ref[...]
    h = jnp.dot(h1_ref[...], w[0:D], preferred_element_type=jnp.float32)
    h = h + jnp.dot(f1_ref[...], w[D:2 * D], preferred_element_type=jnp.float32)
    h = h + jnp.dot(f2, w[2 * D:3 * D], preferred_element_type=jnp.float32)
    h = h + b_ref[...]
    out_ref[...] = jnp.dot(h, wfc_ref[...],
                           preferred_element_type=jnp.float32) + bfc_ref[...]


_mm2 = pl.pallas_call(
    _mm2_body,
    grid=(GRID,),
    in_specs=[
        pl.BlockSpec((RB, D), lambda i: (i, 0)),
        pl.BlockSpec((RB, D), lambda i: (i, 0)),
        pl.BlockSpec((RB, D), lambda i: (i, 0)),
        pl.BlockSpec((RB, D), lambda i: (i, 0)),
        pl.BlockSpec((N, 1), lambda i: (0, 0)),
        pl.BlockSpec(((K + 1) * D, D), lambda i: (0, 0)),
        pl.BlockSpec((1, D), lambda i: (0, 0)),
        pl.BlockSpec((D, D), lambda i: (0, 0)),
        pl.BlockSpec((1, D), lambda i: (0, 0)),
    ],
    out_specs=pl.BlockSpec((RB, D), lambda i: (i, 0)),
    out_shape=jax.ShapeDtypeStruct((N, D), jnp.float32),
)


def kernel(features, edge_index, W1, b1, W2, b2, Wfc, bfc):
    src = edge_index[0].reshape(E // CH, CH)
    dst = edge_index[1].reshape(E // CH, CH)
    zeros_nd = jnp.zeros((NP, D), jnp.float32)
    ones_nd = jnp.ones((N, D), jnp.float32)

    degp = _prop_kernel(ones_nd, src, dst, zeros_nd)
    norm, u0 = _norm_u0(degp[0], degp[1], features)

    p1 = _prop_kernel(u0, src, dst, zeros_nd)
    f1, u1 = _combine(p1[0], p1[1], norm)
    p2 = _prop_kernel(u1, src, dst, zeros_nd)
    h1, u0b = _mm1(features, f1, p2[0], p2[1], norm, W1, b1.reshape(1, D))

    p1b = _prop_kernel(u0b, src, dst, zeros_nd)
    f1b, u1b = _combine(p1b[0], p1b[1], norm)
    p2b = _prop_kernel(u1b, src, dst, zeros_nd)
    out = _mm2(h1, f1b, p2b[0], p2b[1], norm, W2, b2.reshape(1, D),
               Wfc, bfc.reshape(1, D))
    return out


# speedup vs baseline: 1.6661x; 1.6661x over previous
import functools

import jax
import jax.numpy as jnp
from jax import lax
from jax.experimental import pallas as pl
from jax.experimental.pallas import tpu as pltpu
from jax.experimental.pallas import tpu_sc as plsc

N = 10000
D = 128
E = 320000
K = 2

NC = 2
NS = 16
NW = NC * NS
CH = 125
EW = E // NW
CPW = EW // CH
NP = 10240
RPT = NP // NS

RB = 1000
GRID = N // RB

_MESH = plsc.VectorSubcoreMesh(core_axis_name="c", subcore_axis_name="s")


NBUF = 2
GRP = 8


@functools.partial(
    pl.kernel,
    out_type=jax.ShapeDtypeStruct((NC, NP, D), jnp.float32),
    mesh=_MESH,
    scratch_types=[
        pltpu.VMEM((CPW, CH), jnp.int32),
        pltpu.VMEM((2 * GRP, CH), jnp.int32),
        pltpu.VMEM((NBUF, CH, D), jnp.float32),
        pltpu.VMEM_SHARED((NP, D), jnp.float32),
    ] + [pltpu.SemaphoreType.DMA] * 4,
)
def _prop_kernel(u_hbm, src_hbm, dst_hbm, zeros_hbm, out_hbm,
                 dstv, sgrp, buf, acc, *sems):
    isems = sems[:2]
    gsems = sems[2:]
    c = lax.axis_index("c")
    s = lax.axis_index("s")
    w = s * NC + c
    base = w * CPW
    pltpu.sync_copy(dst_hbm.at[pl.ds(base, CPW)], dstv)
    pltpu.sync_copy(zeros_hbm.at[pl.ds(s * RPT, RPT)],
                    acc.at[pl.ds(s * RPT, RPT)])

    def refill(p, g):
        pltpu.async_copy(src_hbm.at[pl.ds(base + g * GRP, GRP)],
                         sgrp.at[pl.ds(p * GRP, GRP)], isems[p])

    def wait_grp(p):
        pltpu.make_async_copy(src_hbm.at[pl.ds(0, GRP)],
                              sgrp.at[pl.ds(p * GRP, GRP)], isems[p]).wait()

    def start_gather(p, r, b):
        pltpu.async_copy(u_hbm.at[sgrp.at[p * GRP + r]], buf.at[b], gsems[b])

    def wait_gather(b):
        pltpu.make_async_copy(u_hbm.at[dstv.at[0]], buf.at[b],
                              gsems[b]).wait()

    def scatter(i, b):
        pltpu.sync_copy(buf.at[b], acc.at[dstv.at[i]], add=True)

    plsc.subcore_barrier()

    refill(0, 0)
    refill(1, 1)
    wait_grp(0)
    start_gather(0, 0, 0)
    start_gather(0, 1, 1)

    def body(gg, carry):
        for k in range(2 * GRP):
            b = k % 2
            i = 2 * GRP * gg + k
            wait_gather(b)
            if k == 6:
                wait_grp(1)
            if k == 14:
                wait_grp(0)
            jk = k + 2
            pj = 0 if (jk < GRP or jk >= 2 * GRP) else 1
            start_gather(pj, jk % GRP, b)
            if k == 7:
                refill(0, 2 * gg + 2)
            if k == 15:
                refill(1, 2 * gg + 3)
            scatter(i, b)
        return carry

    lax.fori_loop(0, CPW // (2 * GRP) - 1, body, 0)

    ebase = CPW - 2 * GRP
    for k in range(2 * GRP):
        b = k % 2
        wait_gather(b)
        if k == 6:
            wait_grp(1)
        if k <= 2 * GRP - 3:
            jk = k + 2
            start_gather(0 if jk < GRP else 1, jk % GRP, b)
        scatter(ebase + k, b)

    plsc.subcore_barrier()
    pltpu.sync_copy(acc.at[pl.ds(s * RPT, RPT)],
                    out_hbm.at[c].at[pl.ds(s * RPT, RPT)])


def _norm_u0_body(dega_ref, degb_ref, feat_ref, norm_ref, u0_ref):
    deg = dega_ref[..., 0:1] + degb_ref[..., 0:1]
    nrm = lax.rsqrt(jnp.maximum(deg, 1.0))
    norm_ref[...] = nrm
    u0_ref[...] = feat_ref[...] * nrm


_norm_u0 = pl.pallas_call(
    _norm_u0_body,
    grid=(GRID,),
    in_specs=[
        pl.BlockSpec((RB, D), lambda i: (i, 0)),
        pl.BlockSpec((RB, D), lambda i: (i, 0)),
        pl.BlockSpec((RB, D), lambda i: (i, 0)),
    ],
    out_specs=[
        pl.BlockSpec((RB, 1), lambda i: (i, 0)),
        pl.BlockSpec((RB, D), lambda i: (i, 0)),
    ],
    out_shape=[
        jax.ShapeDtypeStruct((N, 1), jnp.float32),
        jax.ShapeDtypeStruct((N, D), jnp.float32),
    ],
)


def _combine_body(pa_ref, pb_ref, norm_ref, f_ref, u_ref):
    i = pl.program_id(0)
    nb = norm_ref[pl.ds(i * RB, RB), :]
    f = (pa_ref[...] + pb_ref[...]) * nb
    f_ref[...] = f
    u_ref[...] = f * nb


_combine = pl.pallas_call(
    _combine_body,
    grid=(GRID,),
    in_specs=[
        pl.BlockSpec((RB, D), lambda i: (i, 0)),
        pl.BlockSpec((RB, D), lambda i: (i, 0)),
        pl.BlockSpec((N, 1), lambda i: (0, 0)),
    ],
    out_specs=[
        pl.BlockSpec((RB, D), lambda i: (i, 0)),
        pl.BlockSpec((RB, D), lambda i: (i, 0)),
    ],
    out_shape=[
        jax.ShapeDtypeStruct((N, D), jnp.float32),
        jax.ShapeDtypeStruct((N, D), jnp.float32),
    ],
)


def _mm1_body(f0_ref, f1_ref, pa_ref, pb_ref, norm_ref, w_ref, b_ref,
              h_ref, u_ref):
    i = pl.program_id(0)
    nb = norm_ref[pl.ds(i * RB, RB), :]
    f2 = (pa_ref[...] + pb_ref[...]) * nb
    w = w_ref[...]
    h = jnp.dot(f0_ref[...], w[0:D], preferred_element_type=jnp.float32)
    h = h + jnp.dot(f1_ref[...], w[D:2 * D], preferred_element_type=jnp.float32)
    h = h + jnp.dot(f2, w[2 * D:3 * D], preferred_element_type=jnp.float32)
    h = h + b_ref[...]
    h_ref[...] = h
    u_ref[...] = h * nb


_mm1 = pl.pallas_call(
    _mm1_body,
    grid=(GRID,),
    in_specs=[
        pl.BlockSpec((RB, D), lambda i: (i, 0)),
        pl.BlockSpec((RB, D), lambda i: (i, 0)),
        pl.BlockSpec((RB, D), lambda i: (i, 0)),
        pl.BlockSpec((RB, D), lambda i: (i, 0)),
        pl.BlockSpec((N, 1), lambda i: (0, 0)),
        pl.BlockSpec(((K + 1) * D, D), lambda i: (0, 0)),
        pl.BlockSpec((1, D), lambda i: (0, 0)),
    ],
    out_specs=[
        pl.BlockSpec((RB, D), lambda i: (i, 0)),
        pl.BlockSpec((RB, D), lambda i: (i, 0)),
    ],
    out_shape=[
        jax.ShapeDtypeStruct((N, D), jnp.float32),
        jax.ShapeDtypeStruct((N, D), jnp.float32),
    ],
)


def _mm2_body(h1_ref, f1_ref, pa_ref, pb_ref, norm_ref, w_ref, b_ref,
              wfc_ref, bfc_ref, out_ref):
    i = pl.program_id(0)
    nb = norm_ref[pl.ds(i * RB, RB), :]
    f2 = (pa_ref[...] + pb_ref[...]) * nb
    w = w_ref[...]
    h = jnp.dot(h1_ref[...], w[0:D], preferred_element_type=jnp.float32)
    h = h + jnp.dot(f1_ref[...], w[D:2 * D], preferred_element_type=jnp.float32)
    h = h + jnp.dot(f2, w[2 * D:3 * D], preferred_element_type=jnp.float32)
    h = h + b_ref[...]
    out_ref[...] = jnp.dot(h, wfc_ref[...],
                           preferred_element_type=jnp.float32) + bfc_ref[...]


_mm2 = pl.pallas_call(
    _mm2_body,
    grid=(GRID,),
    in_specs=[
        pl.BlockSpec((RB, D), lambda i: (i, 0)),
        pl.BlockSpec((RB, D), lambda i: (i, 0)),
        pl.BlockSpec((RB, D), lambda i: (i, 0)),
        pl.BlockSpec((RB, D), lambda i: (i, 0)),
        pl.BlockSpec((N, 1), lambda i: (0, 0)),
        pl.BlockSpec(((K + 1) * D, D), lambda i: (0, 0)),
        pl.BlockSpec((1, D), lambda i: (0, 0)),
        pl.BlockSpec((D, D), lambda i: (0, 0)),
        pl.BlockSpec((1, D), lambda i: (0, 0)),
    ],
    out_specs=pl.BlockSpec((RB, D), lambda i: (i, 0)),
    out_shape=jax.ShapeDtypeStruct((N, D), jnp.float32),
)


def kernel(features, edge_index, W1, b1, W2, b2, Wfc, bfc):
    src = edge_index[0].reshape(E // CH, CH)
    dst = edge_index[1].reshape(E // CH, CH)
    zeros_nd = jnp.zeros((NP, D), jnp.float32)
    ones_nd = jnp.ones((N, D), jnp.float32)

    degp = _prop_kernel(ones_nd, src, dst, zeros_nd)
    norm, u0 = _norm_u0(degp[0], degp[1], features)

    p1 = _prop_kernel(u0, src, dst, zeros_nd)
    f1, u1 = _combine(p1[0], p1[1], norm)
    p2 = _prop_kernel(u1, src, dst, zeros_nd)
    h1, u0b = _mm1(features, f1, p2[0], p2[1], norm, W1, b1.reshape(1, D))

    p1b = _prop_kernel(u0b, src, dst, zeros_nd)
    f1b, u1b = _combine(p1b[0], p1b[1], norm)
    p2b = _prop_kernel(u1b, src, dst, zeros_nd)
    out = _mm2(h1, f1b, p2b[0], p2b[1], norm, W2, b2.reshape(1, D),
               Wfc, bfc.reshape(1, D))
    return out


# speedup vs baseline: 1.7326x; 1.0399x over previous
import functools

import jax
import jax.numpy as jnp
from jax import lax
from jax.experimental import pallas as pl
from jax.experimental.pallas import tpu as pltpu
from jax.experimental.pallas import tpu_sc as plsc

N = 10000
D = 128
E = 320000
K = 2

NC = 2
NS = 16
NW = NC * NS
CH = 125
EW = E // NW
CPW = EW // CH
NP = 10240
RPT = NP // NS

RB = 1000
GRID = N // RB

_MESH = plsc.VectorSubcoreMesh(core_axis_name="c", subcore_axis_name="s")


@functools.partial(
    pl.kernel,
    out_type=jax.ShapeDtypeStruct((NC, NP, D), jnp.float32),
    mesh=_MESH,
    scratch_types=[
        pltpu.VMEM((CPW, CH), jnp.int32),
        pltpu.VMEM((128, D), jnp.float32),
        pltpu.VMEM_SHARED((NP, D), jnp.float32),
    ],
)
def _deg_kernel(ones_hbm, dst_hbm, zeros_hbm, out_hbm, dstv, buf, acc):
    c = lax.axis_index("c")
    s = lax.axis_index("s")
    w = s * NC + c
    base = w * CPW
    pltpu.sync_copy(dst_hbm.at[pl.ds(base, CPW)], dstv)
    pltpu.sync_copy(ones_hbm.at[pl.ds(0, 128)], buf)
    pltpu.sync_copy(zeros_hbm.at[pl.ds(s * RPT, RPT)],
                    acc.at[pl.ds(s * RPT, RPT)])
    plsc.subcore_barrier()

    def body(i, carry):
        pltpu.sync_copy(buf.at[pl.ds(0, CH)], acc.at[dstv.at[i]], add=True)
        return carry

    lax.fori_loop(0, CPW, body, 0)
    plsc.subcore_barrier()
    pltpu.sync_copy(acc.at[pl.ds(s * RPT, RPT)],
                    out_hbm.at[c].at[pl.ds(s * RPT, RPT)])


NBUF = 2
GRP = 8


@functools.partial(
    pl.kernel,
    out_type=jax.ShapeDtypeStruct((NC, NP, D), jnp.float32),
    mesh=_MESH,
    scratch_types=[
        pltpu.VMEM((CPW, CH), jnp.int32),
        pltpu.VMEM((2 * GRP, CH), jnp.int32),
        pltpu.VMEM((NBUF, CH, D), jnp.float32),
        pltpu.VMEM_SHARED((NP, D), jnp.float32),
    ] + [pltpu.SemaphoreType.DMA] * 4,
)
def _prop_kernel(u_hbm, src_hbm, dst_hbm, zeros_hbm, out_hbm,
                 dstv, sgrp, buf, acc, *sems):
    isems = sems[:2]
    gsems = sems[2:]
    c = lax.axis_index("c")
    s = lax.axis_index("s")
    w = s * NC + c
    base = w * CPW
    pltpu.sync_copy(dst_hbm.at[pl.ds(base, CPW)], dstv)
    pltpu.sync_copy(zeros_hbm.at[pl.ds(s * RPT, RPT)],
                    acc.at[pl.ds(s * RPT, RPT)])

    def refill(p, g):
        pltpu.async_copy(src_hbm.at[pl.ds(base + g * GRP, GRP)],
                         sgrp.at[pl.ds(p * GRP, GRP)], isems[p])

    def wait_grp(p):
        pltpu.make_async_copy(src_hbm.at[pl.ds(0, GRP)],
                              sgrp.at[pl.ds(p * GRP, GRP)], isems[p]).wait()

    def start_gather(p, r, b):
        pltpu.async_copy(u_hbm.at[sgrp.at[p * GRP + r]], buf.at[b], gsems[b])

    def wait_gather(b):
        pltpu.make_async_copy(u_hbm.at[dstv.at[0]], buf.at[b],
                              gsems[b]).wait()

    def scatter(i, b):
        pltpu.sync_copy(buf.at[b], acc.at[dstv.at[i]], add=True)

    plsc.subcore_barrier()

    refill(0, 0)
    refill(1, 1)
    wait_grp(0)
    start_gather(0, 0, 0)
    start_gather(0, 1, 1)

    def body(gg, carry):
        for k in range(2 * GRP):
            b = k % 2
            i = 2 * GRP * gg + k
            wait_gather(b)
            if k == 6:
                wait_grp(1)
            if k == 14:
                wait_grp(0)
            jk = k + 2
            pj = 0 if (jk < GRP or jk >= 2 * GRP) else 1
            start_gather(pj, jk % GRP, b)
            if k == 7:
                refill(0, 2 * gg + 2)
            if k == 15:
                refill(1, 2 * gg + 3)
            scatter(i, b)
        return carry

    lax.fori_loop(0, CPW // (2 * GRP) - 1, body, 0)

    ebase = CPW - 2 * GRP
    for k in range(2 * GRP):
        b = k % 2
        wait_gather(b)
        if k == 6:
            wait_grp(1)
        if k <= 2 * GRP - 3:
            jk = k + 2
            start_gather(0 if jk < GRP else 1, jk % GRP, b)
        scatter(ebase + k, b)

    plsc.subcore_barrier()
    pltpu.sync_copy(acc.at[pl.ds(s * RPT, RPT)],
                    out_hbm.at[c].at[pl.ds(s * RPT, RPT)])


def _norm_u0_body(dega_ref, degb_ref, feat_ref, norm_ref, u0_ref):
    deg = dega_ref[..., 0:1] + degb_ref[..., 0:1]
    nrm = lax.rsqrt(jnp.maximum(deg, 1.0))
    norm_ref[...] = nrm
    u0_ref[...] = feat_ref[...] * nrm


_norm_u0 = pl.pallas_call(
    _norm_u0_body,
    grid=(GRID,),
    in_specs=[
        pl.BlockSpec((RB, D), lambda i: (i, 0)),
        pl.BlockSpec((RB, D), lambda i: (i, 0)),
        pl.BlockSpec((RB, D), lambda i: (i, 0)),
    ],
    out_specs=[
        pl.BlockSpec((RB, 1), lambda i: (i, 0)),
        pl.BlockSpec((RB, D), lambda i: (i, 0)),
    ],
    out_shape=[
        jax.ShapeDtypeStruct((N, 1), jnp.float32),
        jax.ShapeDtypeStruct((N, D), jnp.float32),
    ],
)


def _combine_body(pa_ref, pb_ref, norm_ref, f_ref, u_ref):
    i = pl.program_id(0)
    nb = norm_ref[pl.ds(i * RB, RB), :]
    f = (pa_ref[...] + pb_ref[...]) * nb
    f_ref[...] = f
    u_ref[...] = f * nb


_combine = pl.pallas_call(
    _combine_body,
    grid=(GRID,),
    in_specs=[
        pl.BlockSpec((RB, D), lambda i: (i, 0)),
        pl.BlockSpec((RB, D), lambda i: (i, 0)),
        pl.BlockSpec((N, 1), lambda i: (0, 0)),
    ],
    out_specs=[
        pl.BlockSpec((RB, D), lambda i: (i, 0)),
        pl.BlockSpec((RB, D), lambda i: (i, 0)),
    ],
    out_shape=[
        jax.ShapeDtypeStruct((N, D), jnp.float32),
        jax.ShapeDtypeStruct((N, D), jnp.float32),
    ],
)


def _mm1_body(f0_ref, f1_ref, pa_ref, pb_ref, norm_ref, w_ref, b_ref,
              h_ref, u_ref):
    i = pl.program_id(0)
    nb = norm_ref[pl.ds(i * RB, RB), :]
    f2 = (pa_ref[...] + pb_ref[...]) * nb
    w = w_ref[...]
    h = jnp.dot(f0_ref[...], w[0:D], preferred_element_type=jnp.float32)
    h = h + jnp.dot(f1_ref[...], w[D:2 * D], preferred_element_type=jnp.float32)
    h = h + jnp.dot(f2, w[2 * D:3 * D], preferred_element_type=jnp.float32)
    h = h + b_ref[...]
    h_ref[...] = h
    u_ref[...] = h * nb


_mm1 = pl.pallas_call(
    _mm1_body,
    grid=(GRID,),
    in_specs=[
        pl.BlockSpec((RB, D), lambda i: (i, 0)),
        pl.BlockSpec((RB, D), lambda i: (i, 0)),
        pl.BlockSpec((RB, D), lambda i: (i, 0)),
        pl.BlockSpec((RB, D), lambda i: (i, 0)),
        pl.BlockSpec((N, 1), lambda i: (0, 0)),
        pl.BlockSpec(((K + 1) * D, D), lambda i: (0, 0)),
        pl.BlockSpec((1, D), lambda i: (0, 0)),
    ],
    out_specs=[
        pl.BlockSpec((RB, D), lambda i: (i, 0)),
        pl.BlockSpec((RB, D), lambda i: (i, 0)),
    ],
    out_shape=[
        jax.ShapeDtypeStruct((N, D), jnp.float32),
        jax.ShapeDtypeStruct((N, D), jnp.float32),
    ],
)


def _mm2_body(h1_ref, f1_ref, pa_ref, pb_ref, norm_ref, w_ref, b_ref,
              wfc_ref, bfc_ref, out_ref):
    i = pl.program_id(0)
    nb = norm_ref[pl.ds(i * RB, RB), :]
    f2 = (pa_ref[...] + pb_ref[...]) * nb
    w = w_ref[...]
    h = jnp.dot(h1_ref[...], w[0:D], preferred_element_type=jnp.float32)
    h = h + jnp.dot(f1_ref[...], w[D:2 * D], preferred_element_type=jnp.float32)
    h = h + jnp.dot(f2, w[2 * D:3 * D], preferred_element_type=jnp.float32)
    h = h + b_ref[...]
    out_ref[...] = jnp.dot(h, wfc_ref[...],
                           preferred_element_type=jnp.float32) + bfc_ref[...]


_mm2 = pl.pallas_call(
    _mm2_body,
    grid=(GRID,),
    in_specs=[
        pl.BlockSpec((RB, D), lambda i: (i, 0)),
        pl.BlockSpec((RB, D), lambda i: (i, 0)),
        pl.BlockSpec((RB, D), lambda i: (i, 0)),
        pl.BlockSpec((RB, D), lambda i: (i, 0)),
        pl.BlockSpec((N, 1), lambda i: (0, 0)),
        pl.BlockSpec(((K + 1) * D, D), lambda i: (0, 0)),
        pl.BlockSpec((1, D), lambda i: (0, 0)),
        pl.BlockSpec((D, D), lambda i: (0, 0)),
        pl.BlockSpec((1, D), lambda i: (0, 0)),
    ],
    out_specs=pl.BlockSpec((RB, D), lambda i: (i, 0)),
    out_shape=jax.ShapeDtypeStruct((N, D), jnp.float32),
)


def kernel(features, edge_index, W1, b1, W2, b2, Wfc, bfc):
    src = edge_index[0].reshape(E // CH, CH)
    dst = edge_index[1].reshape(E // CH, CH)
    zeros_nd = jnp.zeros((NP, D), jnp.float32)
    ones_nd = jnp.ones((N, D), jnp.float32)

    degp = _deg_kernel(ones_nd, dst, zeros_nd)
    norm, u0 = _norm_u0(degp[0], degp[1], features)

    p1 = _prop_kernel(u0, src, dst, zeros_nd)
    f1, u1 = _combine(p1[0], p1[1], norm)
    p2 = _prop_kernel(u1, src, dst, zeros_nd)
    h1, u0b = _mm1(features, f1, p2[0], p2[1], norm, W1, b1.reshape(1, D))

    p1b = _prop_kernel(u0b, src, dst, zeros_nd)
    f1b, u1b = _combine(p1b[0], p1b[1], norm)
    p2b = _prop_kernel(u1b, src, dst, zeros_nd)
    out = _mm2(h1, f1b, p2b[0], p2b[1], norm, W2, b2.reshape(1, D),
               Wfc, bfc.reshape(1, D))
    return out
